# Initial kernel scaffold; baseline (speedup 1.0000x reference)
import jax
import jax.numpy as jnp
from jax.experimental import pallas as pl


def kernel(x, edge_index, weight, bias):
    raise NotImplementedError("write your pallas kernel here")


# speedup vs baseline: 2.0685x; 2.0685x over previous
import functools

import jax
import jax.numpy as jnp
from jax import lax
from jax.experimental import pallas as pl
from jax.experimental.pallas import tpu as pltpu

LANE = 128
TILE = 512


def _masked_log_softmax(logits, num_classes):
    col = lax.broadcasted_iota(jnp.int32, logits.shape, 1)
    logits = jnp.where(col < num_classes, logits, -1e30)
    mx = jnp.max(logits, axis=1, keepdims=True)
    z = logits - mx
    lse = jnp.log(jnp.sum(jnp.exp(z), axis=1, keepdims=True))
    return z - lse


def _linear_kernel(x_ref, w_ref, d_ref, o_ref):
    xb = x_ref[...].astype(jnp.bfloat16)
    xw = jnp.dot(xb, w_ref[...], preferred_element_type=jnp.float32)
    o_ref[...] = (xw * d_ref[...]).astype(o_ref.dtype)


def _prop_kernel(adj_ref, d_ref, m_ref, o_ref, *, tile):
    i = pl.program_id(0)
    a = adj_ref[...].astype(jnp.bfloat16)
    h = jnp.dot(a, m_ref[...], preferred_element_type=jnp.float32)
    start = pl.multiple_of(i * tile, tile)
    h += m_ref[pl.ds(start, tile), :].astype(jnp.float32)
    d = d_ref[...]
    o_ref[...] = (h * (d * d)).astype(o_ref.dtype)


def _prop_final_kernel(adj_ref, d_ref, m_ref, b_ref, o_ref, *, tile,
                       num_classes):
    i = pl.program_id(0)
    a = adj_ref[...].astype(jnp.bfloat16)
    h = jnp.dot(a, m_ref[...], preferred_element_type=jnp.float32)
    start = pl.multiple_of(i * tile, tile)
    h += m_ref[pl.ds(start, tile), :].astype(jnp.float32)
    logits = h * d_ref[...] + b_ref[...]
    o_ref[...] = _masked_log_softmax(logits, num_classes).astype(o_ref.dtype)


def kernel(x, edge_index, weight, bias):
    n, f = x.shape
    c = weight.shape[1]
    row, col = edge_index[0], edge_index[1]
    e = int(row.shape[0])
    t = min(TILE, n)
    cp = ((c + LANE - 1) // LANE) * LANE

    deg = jax.ops.segment_sum(jnp.ones((e,), jnp.float32), row,
                              num_segments=n) + 1.0
    d_inv_sqrt = lax.rsqrt(deg)
    d_p = d_inv_sqrt.reshape(n, 1)

    adj = (jnp.zeros((n, n), jnp.int8)
           .at[row, col].add(jnp.ones((e,), jnp.int8)))

    w_p = jnp.zeros((f, cp), jnp.bfloat16).at[:, :c].set(
        weight.astype(jnp.bfloat16))
    b_p = jnp.zeros((1, cp), jnp.float32).at[0, :c].set(bias)

    grid_rows = n // t
    vmem_limit = 100 * 1024 * 1024

    m0 = pl.pallas_call(
        _linear_kernel,
        out_shape=jax.ShapeDtypeStruct((n, cp), jnp.bfloat16),
        grid=(grid_rows,),
        in_specs=[pl.BlockSpec((t, f), lambda i: (i, 0)),
                  pl.BlockSpec((f, cp), lambda i: (0, 0)),
                  pl.BlockSpec((t, 1), lambda i: (i, 0))],
        out_specs=pl.BlockSpec((t, cp), lambda i: (i, 0)),
        compiler_params=pltpu.CompilerParams(
            dimension_semantics=("parallel",),
            vmem_limit_bytes=vmem_limit),
        cost_estimate=pl.CostEstimate(
            flops=2 * n * f * cp, transcendentals=0,
            bytes_accessed=n * f * 4 + f * cp * 2 + n * 4 + n * cp * 2),
    )(x, w_p, d_p)

    m1 = pl.pallas_call(
        functools.partial(_prop_kernel, tile=t),
        out_shape=jax.ShapeDtypeStruct((n, cp), jnp.bfloat16),
        grid=(grid_rows,),
        in_specs=[pl.BlockSpec((t, n), lambda i: (i, 0)),
                  pl.BlockSpec((t, 1), lambda i: (i, 0)),
                  pl.BlockSpec((n, cp), lambda i: (0, 0))],
        out_specs=pl.BlockSpec((t, cp), lambda i: (i, 0)),
        compiler_params=pltpu.CompilerParams(
            dimension_semantics=("parallel",),
            vmem_limit_bytes=vmem_limit),
        cost_estimate=pl.CostEstimate(
            flops=2 * n * n * cp, transcendentals=0,
            bytes_accessed=n * n + n * cp * 2 + n * 4 + n * cp * 2),
    )(adj, d_p, m0)

    out_p = pl.pallas_call(
        functools.partial(_prop_final_kernel, tile=t, num_classes=c),
        out_shape=jax.ShapeDtypeStruct((n, cp), jnp.float32),
        grid=(grid_rows,),
        in_specs=[pl.BlockSpec((t, n), lambda i: (i, 0)),
                  pl.BlockSpec((t, 1), lambda i: (i, 0)),
                  pl.BlockSpec((n, cp), lambda i: (0, 0)),
                  pl.BlockSpec((1, cp), lambda i: (0, 0))],
        out_specs=pl.BlockSpec((t, cp), lambda i: (i, 0)),
        compiler_params=pltpu.CompilerParams(
            dimension_semantics=("parallel",),
            vmem_limit_bytes=vmem_limit),
        cost_estimate=pl.CostEstimate(
            flops=2 * n * n * cp, transcendentals=n * cp,
            bytes_accessed=n * n + n * cp * 2 + n * 4 + cp * 4 + n * cp * 4),
    )(adj, d_p, m1, b_p)

    return out_p[:, :c]
